# baseline (speedup 1.0000x reference)
import functools

import numpy as np

import jax
import jax.numpy as jnp
from jax import lax
from jax.experimental import pallas as pl
from jax.experimental.pallas import tpu as pltpu
from jax.experimental.pallas import tpu_sc as plsc

ROWS = 128
COLS = 32768
NUM_CORES = 2
NUM_SUBCORES = 16
NUM_WORKERS = NUM_CORES * NUM_SUBCORES
ROWS_PER_WORKER = ROWS // NUM_WORKERS
LANES = 16
NVECS = COLS // LANES
UNROLL = 8
NGROUPS = NVECS // UNROLL

def _vreg_prefix_shift(v, shift_consts):
    s = v
    for idx, msk in shift_consts:
        g = s.at[idx].get(mode="promise_in_bounds")
        s = s + jnp.where(msk, g, jnp.float32(0.0))
    return s


def _inclusive_prefix_tree(ts):
    n = len(ts)
    a = list(ts)
    d = 1
    while d < n:
        for start in range(0, n, 2 * d):
            left_last = a[start + d - 1]
            for j in range(start + d, min(start + 2 * d, n)):
                a[j] = a[j] + left_last
        d *= 2
    return a


def _sc_row_cumsum(x):
    mesh = plsc.VectorSubcoreMesh(
        core_axis_name="c", subcore_axis_name="s")

    @functools.partial(
        pl.kernel,
        out_type=jax.ShapeDtypeStruct((ROWS, COLS), jnp.float32),
        mesh=mesh,
        scratch_types=[
            pltpu.VMEM((2, COLS), jnp.float32),
            pltpu.SemaphoreType.DMA,
            pltpu.SemaphoreType.DMA,
            pltpu.SemaphoreType.DMA,
            pltpu.SemaphoreType.DMA,
        ],
        compiler_params=pltpu.CompilerParams(needs_layout_passes=False),
    )
    def k(x_hbm, out_hbm, buf, in_sem0, in_sem1, out_sem0, out_sem1):
        wid = lax.axis_index("s") * NUM_CORES + lax.axis_index("c")
        iota = lax.iota(jnp.int32, LANES)
        idx_last = jnp.full((LANES,), LANES - 1, jnp.int32)
        shift_consts = [(jnp.maximum(iota - d, 0), iota >= d)
                        for d in (1, 2, 4, 8)]
        in_sems = (in_sem0, in_sem1)
        out_sems = (out_sem0, out_sem1)

        def row_idx(r):
            return wid * ROWS_PER_WORKER + r

        def scan_row(b):
            def group_body(g, c):
                base = g * (UNROLL * LANES)
                sls = [pl.ds(base + j * LANES, LANES) for j in range(UNROLL)]
                ss = [plsc.cumsum(buf[b, sl]) for sl in sls]
                ts = [s.at[idx_last].get(mode="promise_in_bounds")
                      for s in ss]
                incl = _inclusive_prefix_tree(ts)
                pres = [c] + [c + incl[j] for j in range(UNROLL - 1)]
                for j in range(UNROLL):
                    buf[b, sls[j]] = ss[j] + pres[j]
                return c + incl[UNROLL - 1]

            plsc.parallel_loop(
                0, NGROUPS, 1, carry=jnp.zeros((LANES,), jnp.float32)
            )(group_body)

        pending_out = [None, None]
        copy_in = pltpu.async_copy(
            x_hbm.at[row_idx(0)], buf.at[0], in_sems[0])
        for r in range(ROWS_PER_WORKER):
            b = r % 2
            nb = (r + 1) % 2
            if r + 1 < ROWS_PER_WORKER:
                if pending_out[nb] is not None:
                    pending_out[nb].wait()
                    pending_out[nb] = None
                next_in = pltpu.async_copy(
                    x_hbm.at[row_idx(r + 1)], buf.at[nb], in_sems[nb])
            copy_in.wait()
            scan_row(b)
            pending_out[b] = pltpu.async_copy(
                buf.at[b], out_hbm.at[row_idx(r)], out_sems[b])
            if r + 1 < ROWS_PER_WORKER:
                copy_in = next_in
        for p in pending_out:
            if p is not None:
                p.wait()

    return k(x)


BLK = 128
NBLK = COLS // BLK


def _tc_block_scan_kernel(x_ref, o_ref):
    xb = x_ref[0]
    i = lax.broadcasted_iota(jnp.int32, (BLK, BLK), 0)
    j = lax.broadcasted_iota(jnp.int32, (BLK, BLK), 1)
    upper = (i <= j).astype(jnp.float32)
    cum = lax.dot_general(xb, upper, (((1,), (0,)), ((), ())),
                          preferred_element_type=jnp.float32)
    sums = cum[:, BLK - 1:BLK]
    ii = lax.broadcasted_iota(jnp.int32, (NBLK, NBLK), 0)
    jj = lax.broadcasted_iota(jnp.int32, (NBLK, NBLK), 1)
    strict_lower = (jj < ii).astype(jnp.float32)
    carry = lax.dot_general(strict_lower, sums, (((1,), (0,)), ((), ())),
                            preferred_element_type=jnp.float32)
    o_ref[0] = cum + carry


def _tc_row_cumsum(x):
    rows = x.shape[0]
    x3 = x.reshape(rows, NBLK, BLK)
    out = pl.pallas_call(
        _tc_block_scan_kernel,
        grid=(rows,),
        in_specs=[pl.BlockSpec((1, NBLK, BLK), lambda g: (g, 0, 0))],
        out_specs=pl.BlockSpec((1, NBLK, BLK), lambda g: (g, 0, 0)),
        out_shape=jax.ShapeDtypeStruct((rows, NBLK, BLK), jnp.float32),
    )(x3)
    return out.reshape(rows, COLS)


def kernel(x):
    return _tc_row_cumsum(x)


# speedup vs baseline: 1.7610x; 1.7610x over previous
import functools

import numpy as np

import jax
import jax.numpy as jnp
from jax import lax
from jax.experimental import pallas as pl
from jax.experimental.pallas import tpu as pltpu
from jax.experimental.pallas import tpu_sc as plsc

ROWS = 128
COLS = 32768
NUM_CORES = 2
NUM_SUBCORES = 16
NUM_WORKERS = NUM_CORES * NUM_SUBCORES
ROWS_PER_WORKER = ROWS // NUM_WORKERS
LANES = 16
NVECS = COLS // LANES
UNROLL = 8
NGROUPS = NVECS // UNROLL

def _vreg_prefix_shift(v, shift_consts):
    s = v
    for idx, msk in shift_consts:
        g = s.at[idx].get(mode="promise_in_bounds")
        s = s + jnp.where(msk, g, jnp.float32(0.0))
    return s


def _inclusive_prefix_tree(ts):
    n = len(ts)
    a = list(ts)
    d = 1
    while d < n:
        for start in range(0, n, 2 * d):
            left_last = a[start + d - 1]
            for j in range(start + d, min(start + 2 * d, n)):
                a[j] = a[j] + left_last
        d *= 2
    return a


def _sc_row_cumsum(x):
    mesh = plsc.VectorSubcoreMesh(
        core_axis_name="c", subcore_axis_name="s")

    @functools.partial(
        pl.kernel,
        out_type=jax.ShapeDtypeStruct((ROWS, COLS), jnp.float32),
        mesh=mesh,
        scratch_types=[
            pltpu.VMEM((2, COLS), jnp.float32),
            pltpu.SemaphoreType.DMA,
            pltpu.SemaphoreType.DMA,
            pltpu.SemaphoreType.DMA,
            pltpu.SemaphoreType.DMA,
        ],
        compiler_params=pltpu.CompilerParams(needs_layout_passes=False),
    )
    def k(x_hbm, out_hbm, buf, in_sem0, in_sem1, out_sem0, out_sem1):
        wid = lax.axis_index("s") * NUM_CORES + lax.axis_index("c")
        iota = lax.iota(jnp.int32, LANES)
        idx_last = jnp.full((LANES,), LANES - 1, jnp.int32)
        shift_consts = [(jnp.maximum(iota - d, 0), iota >= d)
                        for d in (1, 2, 4, 8)]
        in_sems = (in_sem0, in_sem1)
        out_sems = (out_sem0, out_sem1)

        def row_idx(r):
            return wid * ROWS_PER_WORKER + r

        def scan_row(b):
            def group_body(g, c):
                base = g * (UNROLL * LANES)
                sls = [pl.ds(base + j * LANES, LANES) for j in range(UNROLL)]
                ss = [plsc.cumsum(buf[b, sl]) for sl in sls]
                ts = [s.at[idx_last].get(mode="promise_in_bounds")
                      for s in ss]
                incl = _inclusive_prefix_tree(ts)
                pres = [c] + [c + incl[j] for j in range(UNROLL - 1)]
                for j in range(UNROLL):
                    buf[b, sls[j]] = ss[j] + pres[j]
                return c + incl[UNROLL - 1]

            plsc.parallel_loop(
                0, NGROUPS, 1, carry=jnp.zeros((LANES,), jnp.float32)
            )(group_body)

        pending_out = [None, None]
        copy_in = pltpu.async_copy(
            x_hbm.at[row_idx(0)], buf.at[0], in_sems[0])
        for r in range(ROWS_PER_WORKER):
            b = r % 2
            nb = (r + 1) % 2
            if r + 1 < ROWS_PER_WORKER:
                if pending_out[nb] is not None:
                    pending_out[nb].wait()
                    pending_out[nb] = None
                next_in = pltpu.async_copy(
                    x_hbm.at[row_idx(r + 1)], buf.at[nb], in_sems[nb])
            copy_in.wait()
            scan_row(b)
            pending_out[b] = pltpu.async_copy(
                buf.at[b], out_hbm.at[row_idx(r)], out_sems[b])
            if r + 1 < ROWS_PER_WORKER:
                copy_in = next_in
        for p in pending_out:
            if p is not None:
                p.wait()

    return k(x)


BLK = 128
NBLK = COLS // BLK


TC_ROWS_PER_STEP = 4


def _tc_block_scan_kernel(x_ref, upper_ref, strict_ref, o_ref):
    xb = x_ref[...]
    cum = lax.dot_general(xb, upper_ref[...], (((2,), (0,)), ((), ())),
                          preferred_element_type=jnp.float32)
    sums = cum[:, :, BLK - 1]
    carry = lax.dot_general(sums, strict_ref[...], (((1,), (0,)), ((), ())),
                            preferred_element_type=jnp.float32)
    o_ref[...] = cum + carry[:, :, None]


def _tc_row_cumsum(x):
    rows = x.shape[0]
    x3 = x.reshape(rows, NBLK, BLK)
    i = lax.broadcasted_iota(jnp.int32, (BLK, BLK), 0)
    j = lax.broadcasted_iota(jnp.int32, (BLK, BLK), 1)
    upper = (i <= j).astype(jnp.float32)
    ii = lax.broadcasted_iota(jnp.int32, (NBLK, NBLK), 0)
    jj = lax.broadcasted_iota(jnp.int32, (NBLK, NBLK), 1)
    strict_upper = (ii < jj).astype(jnp.float32)
    out = pl.pallas_call(
        _tc_block_scan_kernel,
        grid=(rows // TC_ROWS_PER_STEP,),
        in_specs=[
            pl.BlockSpec((TC_ROWS_PER_STEP, NBLK, BLK),
                         lambda g: (g, 0, 0)),
            pl.BlockSpec((BLK, BLK), lambda g: (0, 0)),
            pl.BlockSpec((NBLK, NBLK), lambda g: (0, 0)),
        ],
        out_specs=pl.BlockSpec((TC_ROWS_PER_STEP, NBLK, BLK),
                               lambda g: (g, 0, 0)),
        out_shape=jax.ShapeDtypeStruct((rows, NBLK, BLK), jnp.float32),
    )(x3, upper, strict_upper)
    return out.reshape(rows, COLS)


def kernel(x):
    return _tc_row_cumsum(x)


# speedup vs baseline: 2.0287x; 1.1520x over previous
import functools

import numpy as np

import jax
import jax.numpy as jnp
from jax import lax
from jax.experimental import pallas as pl
from jax.experimental.pallas import tpu as pltpu
from jax.experimental.pallas import tpu_sc as plsc

ROWS = 128
COLS = 32768
NUM_CORES = 2
NUM_SUBCORES = 16
NUM_WORKERS = NUM_CORES * NUM_SUBCORES
ROWS_PER_WORKER = ROWS // NUM_WORKERS
LANES = 16
NVECS = COLS // LANES
UNROLL = 8
NGROUPS = NVECS // UNROLL

def _vreg_prefix_shift(v, shift_consts):
    s = v
    for idx, msk in shift_consts:
        g = s.at[idx].get(mode="promise_in_bounds")
        s = s + jnp.where(msk, g, jnp.float32(0.0))
    return s


def _inclusive_prefix_tree(ts):
    n = len(ts)
    a = list(ts)
    d = 1
    while d < n:
        for start in range(0, n, 2 * d):
            left_last = a[start + d - 1]
            for j in range(start + d, min(start + 2 * d, n)):
                a[j] = a[j] + left_last
        d *= 2
    return a


def _sc_row_cumsum(x):
    mesh = plsc.VectorSubcoreMesh(
        core_axis_name="c", subcore_axis_name="s")

    @functools.partial(
        pl.kernel,
        out_type=jax.ShapeDtypeStruct((ROWS, COLS), jnp.float32),
        mesh=mesh,
        scratch_types=[
            pltpu.VMEM((2, COLS), jnp.float32),
            pltpu.SemaphoreType.DMA,
            pltpu.SemaphoreType.DMA,
            pltpu.SemaphoreType.DMA,
            pltpu.SemaphoreType.DMA,
        ],
        compiler_params=pltpu.CompilerParams(needs_layout_passes=False),
    )
    def k(x_hbm, out_hbm, buf, in_sem0, in_sem1, out_sem0, out_sem1):
        wid = lax.axis_index("s") * NUM_CORES + lax.axis_index("c")
        iota = lax.iota(jnp.int32, LANES)
        idx_last = jnp.full((LANES,), LANES - 1, jnp.int32)
        shift_consts = [(jnp.maximum(iota - d, 0), iota >= d)
                        for d in (1, 2, 4, 8)]
        in_sems = (in_sem0, in_sem1)
        out_sems = (out_sem0, out_sem1)

        def row_idx(r):
            return wid * ROWS_PER_WORKER + r

        def scan_row(b):
            def group_body(g, c):
                base = g * (UNROLL * LANES)
                sls = [pl.ds(base + j * LANES, LANES) for j in range(UNROLL)]
                ss = [plsc.cumsum(buf[b, sl]) for sl in sls]
                ts = [s.at[idx_last].get(mode="promise_in_bounds")
                      for s in ss]
                incl = _inclusive_prefix_tree(ts)
                pres = [c] + [c + incl[j] for j in range(UNROLL - 1)]
                for j in range(UNROLL):
                    buf[b, sls[j]] = ss[j] + pres[j]
                return c + incl[UNROLL - 1]

            plsc.parallel_loop(
                0, NGROUPS, 1, carry=jnp.zeros((LANES,), jnp.float32)
            )(group_body)

        pending_out = [None, None]
        copy_in = pltpu.async_copy(
            x_hbm.at[row_idx(0)], buf.at[0], in_sems[0])
        for r in range(ROWS_PER_WORKER):
            b = r % 2
            nb = (r + 1) % 2
            if r + 1 < ROWS_PER_WORKER:
                if pending_out[nb] is not None:
                    pending_out[nb].wait()
                    pending_out[nb] = None
                next_in = pltpu.async_copy(
                    x_hbm.at[row_idx(r + 1)], buf.at[nb], in_sems[nb])
            copy_in.wait()
            scan_row(b)
            pending_out[b] = pltpu.async_copy(
                buf.at[b], out_hbm.at[row_idx(r)], out_sems[b])
            if r + 1 < ROWS_PER_WORKER:
                copy_in = next_in
        for p in pending_out:
            if p is not None:
                p.wait()

    return k(x)


BLK = 128
NBLK = COLS // BLK


TC_ROWS_PER_STEP = 8


def _tc_block_scan_kernel(x_ref, upper_ref, strict_ref, o_ref):
    xb = x_ref[...]
    cum = lax.dot_general(xb, upper_ref[...], (((2,), (0,)), ((), ())),
                          preferred_element_type=jnp.float32)
    sums = cum[:, :, BLK - 1]
    carry = lax.dot_general(sums, strict_ref[...], (((1,), (0,)), ((), ())),
                            preferred_element_type=jnp.float32)
    o_ref[...] = cum + carry[:, :, None]


def _tc_row_cumsum(x):
    rows = x.shape[0]
    x3 = x.reshape(rows, NBLK, BLK)
    i = lax.broadcasted_iota(jnp.int32, (BLK, BLK), 0)
    j = lax.broadcasted_iota(jnp.int32, (BLK, BLK), 1)
    upper = (i <= j).astype(jnp.float32)
    ii = lax.broadcasted_iota(jnp.int32, (NBLK, NBLK), 0)
    jj = lax.broadcasted_iota(jnp.int32, (NBLK, NBLK), 1)
    strict_upper = (ii < jj).astype(jnp.float32)
    out = pl.pallas_call(
        _tc_block_scan_kernel,
        grid=(rows // TC_ROWS_PER_STEP,),
        in_specs=[
            pl.BlockSpec((TC_ROWS_PER_STEP, NBLK, BLK),
                         lambda g: (g, 0, 0)),
            pl.BlockSpec((BLK, BLK), lambda g: (0, 0)),
            pl.BlockSpec((NBLK, NBLK), lambda g: (0, 0)),
        ],
        out_specs=pl.BlockSpec((TC_ROWS_PER_STEP, NBLK, BLK),
                               lambda g: (g, 0, 0)),
        out_shape=jax.ShapeDtypeStruct((rows, NBLK, BLK), jnp.float32),
    )(x3, upper, strict_upper)
    return out.reshape(rows, COLS)


def kernel(x):
    return _tc_row_cumsum(x)


# speedup vs baseline: 5.3834x; 2.6536x over previous
import functools

import numpy as np

import jax
import jax.numpy as jnp
from jax import lax
from jax.experimental import pallas as pl
from jax.experimental.pallas import tpu as pltpu
from jax.experimental.pallas import tpu_sc as plsc

ROWS = 128
COLS = 32768
NUM_CORES = 2
NUM_SUBCORES = 16
NUM_WORKERS = NUM_CORES * NUM_SUBCORES
ROWS_PER_WORKER = ROWS // NUM_WORKERS
LANES = 16
NVECS = COLS // LANES
UNROLL = 8
NGROUPS = NVECS // UNROLL

def _vreg_prefix_shift(v, shift_consts):
    s = v
    for idx, msk in shift_consts:
        g = s.at[idx].get(mode="promise_in_bounds")
        s = s + jnp.where(msk, g, jnp.float32(0.0))
    return s


def _inclusive_prefix_tree(ts):
    n = len(ts)
    a = list(ts)
    d = 1
    while d < n:
        for start in range(0, n, 2 * d):
            left_last = a[start + d - 1]
            for j in range(start + d, min(start + 2 * d, n)):
                a[j] = a[j] + left_last
        d *= 2
    return a


def _sc_row_cumsum(x):
    mesh = plsc.VectorSubcoreMesh(
        core_axis_name="c", subcore_axis_name="s")

    @functools.partial(
        pl.kernel,
        out_type=jax.ShapeDtypeStruct((ROWS, COLS), jnp.float32),
        mesh=mesh,
        scratch_types=[
            pltpu.VMEM((2, COLS), jnp.float32),
            pltpu.SemaphoreType.DMA,
            pltpu.SemaphoreType.DMA,
            pltpu.SemaphoreType.DMA,
            pltpu.SemaphoreType.DMA,
        ],
        compiler_params=pltpu.CompilerParams(needs_layout_passes=False),
    )
    def k(x_hbm, out_hbm, buf, in_sem0, in_sem1, out_sem0, out_sem1):
        wid = lax.axis_index("s") * NUM_CORES + lax.axis_index("c")
        iota = lax.iota(jnp.int32, LANES)
        idx_last = jnp.full((LANES,), LANES - 1, jnp.int32)
        shift_consts = [(jnp.maximum(iota - d, 0), iota >= d)
                        for d in (1, 2, 4, 8)]
        in_sems = (in_sem0, in_sem1)
        out_sems = (out_sem0, out_sem1)

        def row_idx(r):
            return wid * ROWS_PER_WORKER + r

        def scan_row(b):
            def group_body(g, c):
                base = g * (UNROLL * LANES)
                sls = [pl.ds(base + j * LANES, LANES) for j in range(UNROLL)]
                ss = [plsc.cumsum(buf[b, sl]) for sl in sls]
                ts = [s.at[idx_last].get(mode="promise_in_bounds")
                      for s in ss]
                incl = _inclusive_prefix_tree(ts)
                pres = [c] + [c + incl[j] for j in range(UNROLL - 1)]
                for j in range(UNROLL):
                    buf[b, sls[j]] = ss[j] + pres[j]
                return c + incl[UNROLL - 1]

            plsc.parallel_loop(
                0, NGROUPS, 1, carry=jnp.zeros((LANES,), jnp.float32)
            )(group_body)

        pending_out = [None, None]
        copy_in = pltpu.async_copy(
            x_hbm.at[row_idx(0)], buf.at[0], in_sems[0])
        for r in range(ROWS_PER_WORKER):
            b = r % 2
            nb = (r + 1) % 2
            if r + 1 < ROWS_PER_WORKER:
                if pending_out[nb] is not None:
                    pending_out[nb].wait()
                    pending_out[nb] = None
                next_in = pltpu.async_copy(
                    x_hbm.at[row_idx(r + 1)], buf.at[nb], in_sems[nb])
            copy_in.wait()
            scan_row(b)
            pending_out[b] = pltpu.async_copy(
                buf.at[b], out_hbm.at[row_idx(r)], out_sems[b])
            if r + 1 < ROWS_PER_WORKER:
                copy_in = next_in
        for p in pending_out:
            if p is not None:
                p.wait()

    return k(x)


BLK = 128
NBLK = COLS // BLK


TC_COL_BLK = 2048
TC_SUB = TC_COL_BLK // BLK


def _tc_block_scan_kernel(x_ref, upper_ref, o_ref, carry_ref):
    rows = x_ref.shape[0]

    @pl.when(pl.program_id(0) == 0)
    def _init():
        carry_ref[...] = jnp.zeros((rows, BLK), jnp.float32)

    xs = x_ref[...]
    u = upper_ref[...]
    c = carry_ref[...]
    outs = []
    for i in range(TC_SUB):
        blk = xs[:, i * BLK:(i + 1) * BLK]
        cum = lax.dot_general(blk, u, (((1,), (0,)), ((), ())),
                              preferred_element_type=jnp.float32) + c
        outs.append(cum)
        c = jnp.broadcast_to(cum[:, BLK - 1:BLK], (rows, BLK))
    o_ref[...] = jnp.concatenate(outs, axis=1)
    carry_ref[...] = c


def _tc_row_cumsum(x):
    rows = x.shape[0]
    i = lax.broadcasted_iota(jnp.int32, (BLK, BLK), 0)
    j = lax.broadcasted_iota(jnp.int32, (BLK, BLK), 1)
    upper = (i <= j).astype(jnp.float32)
    return pl.pallas_call(
        _tc_block_scan_kernel,
        grid=(COLS // TC_COL_BLK,),
        in_specs=[
            pl.BlockSpec((rows, TC_COL_BLK), lambda g: (0, g)),
            pl.BlockSpec((BLK, BLK), lambda g: (0, 0)),
        ],
        out_specs=pl.BlockSpec((rows, TC_COL_BLK), lambda g: (0, g)),
        out_shape=jax.ShapeDtypeStruct((rows, COLS), jnp.float32),
        scratch_shapes=[pltpu.VMEM((rows, BLK), jnp.float32)],
    )(x, upper)


def kernel(x):
    return _tc_row_cumsum(x)
